# baseline (speedup 1.0000x reference)
import functools
import math

import jax
import jax.numpy as jnp
import numpy as np
from jax import lax
from jax.experimental import pallas as pl
from jax.experimental.pallas import tpu as pltpu
from jax.experimental.pallas import tpu_sc as plsc

N = 10000
E = 160000
D = 384
H = 12
DH = 32
DFF = 4 * D
NP = 10240
BLK = 256
GRID = NP // BLK

NW = 32
EPW = E // NW
NPW = NP // NW
_SENT = np.uint32(0xFFFFFFFF)

_INTERPRET = False


def _lane_shift_by(x, r):
    idx = jnp.minimum(lax.iota(jnp.int32, 16) + r, 15)
    return lax.gather(
        x, idx[:, None],
        lax.GatherDimensionNumbers(offset_dims=(), collapsed_slice_dims=(0,),
                                   start_index_map=(0,)),
        (1,), mode=lax.GatherScatterMode.PROMISE_IN_BOUNDS)


def _sc_winner_partials(dst_hbm, part_hbm, dstv, wloc, sem):
    wid = lax.axis_index("s") * 2 + lax.axis_index("c")
    base = wid * EPW
    pltpu.sync_copy(dst_hbm.at[pl.ds(base * 1, EPW)], dstv)

    def init_body(i, _):
        st = pl.multiple_of(i * 16, 16)
        wloc[pl.ds(st, 16)] = jnp.full((16,), -1, jnp.int32)
        return 0
    lax.fori_loop(0, NP // 16, init_body, 0)

    iota = lax.iota(jnp.int32, 16)
    nvec = (EPW + 15) // 16

    def body(i, _):
        st = pl.multiple_of(i * 16, 16)
        d16 = dstv[pl.ds(st, 16)]
        lane_e = i * 16 + iota
        valid = lane_e < EPW
        es = base + lane_e
        conflict = iota < 0
        for r in range(1, 16):
            shifted = _lane_shift_by(d16, r)
            conflict = conflict | ((shifted == d16) & (iota < 16 - r))
        ok = valid & jnp.logical_not(conflict)
        plsc.store_scatter(wloc, [d16], es, mask=ok)
        return 0
    lax.fori_loop(0, nvec, body, 0)
    handles = []
    for s in range(NW):
        handles.append(pltpu.async_copy(
            wloc.at[pl.ds(s * NPW, NPW)],
            part_hbm.at[pl.ds((s * NW + wid) * NPW, NPW)], sem))
    for hd in handles:
        hd.wait()


def _sc_merge_gather(part_hbm, src_hbm, h_hbm, m_hbm, hg_hbm,
                     pblk, winc, srcv, mbuf, bufa, bufb, sema, semb):
    wid = lax.axis_index("s") * 2 + lax.axis_index("c")
    nbase = wid * NPW
    pltpu.sync_copy(part_hbm.at[pl.ds(wid * (NW * NPW), NW * NPW)], pblk)

    def merge_body(j, _):
        st = pl.multiple_of(j * 16, 16)
        mx = pblk[pl.ds(st, 16)]
        for r in range(1, NW):
            mx = jnp.maximum(mx, pblk[pl.ds(r * NPW + st, 16)])
        winc[pl.ds(st, 16)] = jnp.maximum(mx, 0)
        mbuf[pl.ds(st, 16)] = jnp.where(mx >= 0, 1.0, 0.0).astype(jnp.float32)
        return 0
    lax.fori_loop(0, NPW // 16, merge_body, 0)
    pltpu.sync_copy(mbuf, m_hbm.at[pl.ds(nbase * 1, NPW)])
    pltpu.async_copy(src_hbm.at[winc], srcv, sema).wait()

    ck = NPW // 4
    cps = {}
    cps[0] = pltpu.async_copy(h_hbm.at[srcv.at[pl.ds(0 * ck, ck)]], bufa, sema)
    cps[1] = pltpu.async_copy(h_hbm.at[srcv.at[pl.ds(1 * ck, ck)]], bufb, semb)
    for c in range(4):
        buf = bufa if c % 2 == 0 else bufb
        sem = sema if c % 2 == 0 else semb
        cps[c].wait()
        pltpu.sync_copy(buf, hg_hbm.at[pl.ds(nbase + c * ck, ck)])
        if c + 2 < 4:
            cps[c + 2] = pltpu.async_copy(
                h_hbm.at[srcv.at[pl.ds((c + 2) * ck, ck)]], buf, sem)


def _tca_body(feat_ref, fcw_ref, b2_ref, h_ref):
    f = feat_ref[...]
    h = jnp.dot(f, fcw_ref[...], preferred_element_type=jnp.float32)
    h_ref[...] = h + b2_ref[0:1, :]


def _bf(x):
    return x.astype(jnp.bfloat16)


def _tcb_body(h_ref, hg_ref, m_ref, qwt_ref, kwt_ref, vwt_ref, bt_ref,
              b8_ref, fb1_ref, cstt_ref, w1_ref, w2_ref, out_ref, hpt_ref):
    hb = h_ref[...]
    hg = hg_ref[...]
    ht = hb.T
    hgt = hg.T
    htb = _bf(ht)
    hgtb = _bf(hgt)
    qt = jnp.dot(qwt_ref[...], htb, preferred_element_type=jnp.float32)
    qt = qt + bt_ref[:, 0:1]
    kt = jnp.dot(kwt_ref[...], hgtb, preferred_element_type=jnp.float32)
    kt = kt + bt_ref[:, 1:2]
    vt = jnp.dot(vwt_ref[...], hgtb, preferred_element_type=jnp.float32)
    vt = vt + bt_ref[:, 2:3]

    eff = cstt_ref[13:14, 0:1]
    res = cstt_ref[13:14, 1:2]

    for hh in range(H):
        qh = qt[hh * DH:(hh + 1) * DH, :]
        ss = []
        for g in range(H):
            kg = kt[g * DH:(g + 1) * DH, :]
            ss.append(jnp.sum(qh * kg, axis=0, keepdims=True))
        s = jnp.concatenate(ss, axis=0)
        s = s * eff + cstt_ref[0:H, hh:hh + 1]
        smax = jnp.max(s, axis=0, keepdims=True)
        e = jnp.exp(s - smax)
        rs = 1.0 / jnp.sum(e, axis=0, keepdims=True)
        oh = e[0:1, :] * vt[0:DH, :]
        for g in range(1, H):
            oh = oh + e[g:g + 1, :] * vt[g * DH:(g + 1) * DH, :]
        hpt_ref[hh * DH:(hh + 1) * DH, :] = oh * rs
    h_part = hpt_ref[...].T

    m = m_ref[...]
    h_upd = h_part * m + hb * (1.0 - m)
    x1 = hb + h_upd * res
    mu = jnp.mean(x1, axis=1, keepdims=True)
    var = jnp.mean((x1 - mu) ** 2, axis=1, keepdims=True)
    x = (x1 - mu) / jnp.sqrt(var + 1e-5) * b8_ref[2:3, :] + b8_ref[3:4, :]
    t = jnp.dot(_bf(x), w1_ref[...], preferred_element_type=jnp.float32)
    t = jnp.maximum(t + fb1_ref[0:1, :], 0.0)
    y = jnp.dot(_bf(t), w2_ref[...], preferred_element_type=jnp.float32)
    y = y + b8_ref[6:7, :]
    x2 = x + y * res
    mu2 = jnp.mean(x2, axis=1, keepdims=True)
    var2 = jnp.mean((x2 - mu2) ** 2, axis=1, keepdims=True)
    out_ref[...] = ((x2 - mu2) / jnp.sqrt(var2 + 1e-5) * b8_ref[4:5, :]
                    + b8_ref[5:6, :])


def _row_spec():
    return pl.BlockSpec((BLK, D), lambda i: (i, 0))


def _full_spec(shape):
    nd = len(shape)
    return pl.BlockSpec(shape, lambda i: (0,) * nd)


def kernel(feat, edge_index, fc_w, fc_b, q_w, q_b, k_w, k_b, v_w, v_b,
           attn_scale, attn_bias, res_scale, ln1_g, ln1_b, ln2_g, ln2_b,
           ffn_w1, ffn_b1, ffn_w2, ffn_b2):
    feat_p = jnp.pad(feat, ((0, NP - N), (0, 0)))
    b2 = jnp.zeros((8, D), jnp.float32).at[0].set(fc_b)

    h_p = pl.pallas_call(
        _tca_body,
        grid=(GRID,),
        in_specs=[_row_spec(), _full_spec((D, D)), _full_spec((8, D))],
        out_specs=_row_spec(),
        out_shape=jax.ShapeDtypeStruct((NP, D), jnp.float32),
        interpret=_INTERPRET,
    )(feat_p.astype(jnp.bfloat16), fc_w.astype(jnp.bfloat16), b2)

    src = edge_index[0]
    dst = edge_index[1]
    mesh = plsc.VectorSubcoreMesh(core_axis_name="c", subcore_axis_name="s")

    scp = pltpu.CompilerParams(needs_layout_passes=False)
    sc1 = functools.partial(
        pl.kernel, mesh=mesh, compiler_params=scp,
        out_type=jax.ShapeDtypeStruct((NW * NP,), jnp.int32),
        scratch_types=[pltpu.VMEM((EPW,), jnp.int32),
                       pltpu.VMEM((NP,), jnp.int32),
                       pltpu.SemaphoreType.DMA],
    )(_sc_winner_partials)
    partials = sc1(dst)

    sc2 = functools.partial(
        pl.kernel, mesh=mesh, compiler_params=scp,
        out_type=[jax.ShapeDtypeStruct((NP,), jnp.float32),
                  jax.ShapeDtypeStruct((NP, D), jnp.float32)],
        scratch_types=[pltpu.VMEM((NW * NPW,), jnp.int32),
                       pltpu.VMEM((NPW,), jnp.int32),
                       pltpu.VMEM((NPW,), jnp.int32),
                       pltpu.VMEM((NPW,), jnp.float32),
                       pltpu.VMEM((NPW // 4, D), jnp.float32),
                       pltpu.VMEM((NPW // 4, D), jnp.float32),
                       pltpu.SemaphoreType.DMA,
                       pltpu.SemaphoreType.DMA],
    )(_sc_merge_gather)
    m1d, hg = sc2(partials, src, h_p)
    m = m1d.reshape(NP, 1)

    bt = jnp.zeros((D, 8), jnp.float32)
    bt = bt.at[:, 0].set(q_b).at[:, 1].set(k_b).at[:, 2].set(v_b)
    b8 = (jnp.zeros((8, D), jnp.float32)
          .at[2].set(ln1_g).at[3].set(ln1_b)
          .at[4].set(ln2_g).at[5].set(ln2_b)
          .at[6].set(ffn_b2))
    fb1 = jnp.zeros((8, DFF), jnp.float32).at[0].set(ffn_b1)
    cstt = jnp.zeros((16, 128), jnp.float32)
    cstt = cstt.at[0:H, 0:H].set(attn_bias[:, :, 0].T)
    cstt = cstt.at[13, 0].set(attn_scale / math.sqrt(DH))
    cstt = cstt.at[13, 1].set(res_scale)

    out_p = pl.pallas_call(
        _tcb_body,
        grid=(GRID,),
        in_specs=[_row_spec(), _row_spec(),
                  pl.BlockSpec((BLK, 1), lambda i: (i, 0)),
                  _full_spec((D, D)), _full_spec((D, D)), _full_spec((D, D)),
                  _full_spec((D, 8)),
                  _full_spec((8, D)), _full_spec((8, DFF)),
                  _full_spec((16, 128)),
                  _full_spec((D, DFF)), _full_spec((DFF, D))],
        out_specs=_row_spec(),
        out_shape=jax.ShapeDtypeStruct((NP, D), jnp.float32),
        scratch_shapes=[pltpu.VMEM((D, BLK), jnp.float32)],
        interpret=_INTERPRET,
    )(h_p, hg, m, q_w.T.astype(jnp.bfloat16), k_w.T.astype(jnp.bfloat16),
      v_w.T.astype(jnp.bfloat16), bt, b8, fb1, cstt,
      ffn_w1.astype(jnp.bfloat16), ffn_w2.astype(jnp.bfloat16))

    return out_p[:N]


# speedup vs baseline: 1.0234x; 1.0234x over previous
import functools
import math

import jax
import jax.numpy as jnp
import numpy as np
from jax import lax
from jax.experimental import pallas as pl
from jax.experimental.pallas import tpu as pltpu
from jax.experimental.pallas import tpu_sc as plsc

N = 10000
E = 160000
D = 384
H = 12
DH = 32
DFF = 4 * D
NP = 10240
BLK = 256
GRID = NP // BLK

NW = 32
EPW = E // NW
NPW = NP // NW
_SENT = np.uint32(0xFFFFFFFF)

_INTERPRET = False


def _lane_shift_by(x, r):
    idx = jnp.minimum(lax.iota(jnp.int32, 16) + r, 15)
    return lax.gather(
        x, idx[:, None],
        lax.GatherDimensionNumbers(offset_dims=(), collapsed_slice_dims=(0,),
                                   start_index_map=(0,)),
        (1,), mode=lax.GatherScatterMode.PROMISE_IN_BOUNDS)


def _sc_winner_partials(dst_hbm, part_hbm, dstv, wloc, sem):
    wid = lax.axis_index("s") * 2 + lax.axis_index("c")
    base = wid * EPW
    pltpu.sync_copy(dst_hbm.at[pl.ds(base * 1, EPW)], dstv)

    def init_body(i, _):
        st = pl.multiple_of(i * 16, 16)
        wloc[pl.ds(st, 16)] = jnp.full((16,), -1, jnp.int32)
        return 0
    lax.fori_loop(0, NP // 16, init_body, 0)

    iota = lax.iota(jnp.int32, 16)
    nvec = (EPW + 15) // 16

    def body(i, _):
        st = pl.multiple_of(i * 16, 16)
        d16 = dstv[pl.ds(st, 16)]
        lane_e = i * 16 + iota
        valid = lane_e < EPW
        es = base + lane_e
        conflict = iota < 0
        for r in range(1, 16):
            shifted = _lane_shift_by(d16, r)
            conflict = conflict | ((shifted == d16) & (iota < 16 - r))
        ok = valid & jnp.logical_not(conflict)
        plsc.store_scatter(wloc, [d16], es, mask=ok)
        return 0
    lax.fori_loop(0, nvec, body, 0)
    handles = []
    for s in range(NW):
        handles.append(pltpu.async_copy(
            wloc.at[pl.ds(s * NPW, NPW)],
            part_hbm.at[pl.ds((s * NW + wid) * NPW, NPW)], sem))
    for hd in handles:
        hd.wait()


def _sc_merge_gather(part_hbm, src_hbm, h_hbm, m_hbm, hg_hbm,
                     pblk, winc, srcv, mbuf, bufa, bufb, sema, semb):
    wid = lax.axis_index("s") * 2 + lax.axis_index("c")
    nbase = wid * NPW
    pltpu.sync_copy(part_hbm.at[pl.ds(wid * (NW * NPW), NW * NPW)], pblk)

    def merge_body(j, _):
        st = pl.multiple_of(j * 16, 16)
        mx = pblk[pl.ds(st, 16)]
        for r in range(1, NW):
            mx = jnp.maximum(mx, pblk[pl.ds(r * NPW + st, 16)])
        winc[pl.ds(st, 16)] = jnp.maximum(mx, 0)
        mbuf[pl.ds(st, 16)] = jnp.where(mx >= 0, 1.0, 0.0).astype(jnp.float32)
        return 0
    lax.fori_loop(0, NPW // 16, merge_body, 0)
    pltpu.sync_copy(mbuf, m_hbm.at[pl.ds(nbase * 1, NPW)])
    pltpu.async_copy(src_hbm.at[winc], srcv, sema).wait()

    ck = NPW // 4
    cps = {}
    cps[0] = pltpu.async_copy(h_hbm.at[srcv.at[pl.ds(0 * ck, ck)]], bufa, sema)
    cps[1] = pltpu.async_copy(h_hbm.at[srcv.at[pl.ds(1 * ck, ck)]], bufb, semb)
    for c in range(4):
        buf = bufa if c % 2 == 0 else bufb
        sem = sema if c % 2 == 0 else semb
        cps[c].wait()
        pltpu.sync_copy(buf, hg_hbm.at[pl.ds(nbase + c * ck, ck)])
        if c + 2 < 4:
            cps[c + 2] = pltpu.async_copy(
                h_hbm.at[srcv.at[pl.ds((c + 2) * ck, ck)]], buf, sem)


def _tca_body(feat_ref, fcw_ref, b2_ref, h_ref):
    f = feat_ref[...]
    h = jnp.dot(f, fcw_ref[...], preferred_element_type=jnp.float32)
    h_ref[...] = h + b2_ref[0:1, :]


def _bf(x):
    return x.astype(jnp.bfloat16)


def _tcb_body(h_ref, hg_ref, m_ref, qwt_ref, kwt_ref, vwt_ref, bt_ref,
              b8_ref, fb1_ref, cstt_ref, w1_ref, w2_ref, out_ref, hpt_ref):
    hb = h_ref[...]
    hg = hg_ref[...]
    htb = _bf(hb)
    hgtb = _bf(hg)
    dnt = (((1,), (1,)), ((), ()))
    qt = lax.dot_general(qwt_ref[...], htb, dnt,
                         preferred_element_type=jnp.float32)
    qt = qt + bt_ref[:, 0:1]
    kt = lax.dot_general(kwt_ref[...], hgtb, dnt,
                         preferred_element_type=jnp.float32)
    kt = kt + bt_ref[:, 1:2]
    vt = lax.dot_general(vwt_ref[...], hgtb, dnt,
                         preferred_element_type=jnp.float32)
    vt = vt + bt_ref[:, 2:3]

    eff = cstt_ref[13:14, 0:1]
    res = cstt_ref[13:14, 1:2]

    for hh in range(H):
        qh = qt[hh * DH:(hh + 1) * DH, :]
        ss = []
        for g in range(H):
            kg = kt[g * DH:(g + 1) * DH, :]
            ss.append(jnp.sum(qh * kg, axis=0, keepdims=True))
        s = jnp.concatenate(ss, axis=0)
        s = s * eff + cstt_ref[0:H, hh:hh + 1]
        smax = jnp.max(s, axis=0, keepdims=True)
        e = jnp.exp(s - smax)
        rs = 1.0 / jnp.sum(e, axis=0, keepdims=True)
        oh = e[0:1, :] * vt[0:DH, :]
        for g in range(1, H):
            oh = oh + e[g:g + 1, :] * vt[g * DH:(g + 1) * DH, :]
        hpt_ref[hh * DH:(hh + 1) * DH, :] = oh * rs
    h_part = hpt_ref[...].T

    m = m_ref[...]
    h_upd = h_part * m + hb * (1.0 - m)
    x1 = hb + h_upd * res
    mu = jnp.mean(x1, axis=1, keepdims=True)
    var = jnp.mean((x1 - mu) ** 2, axis=1, keepdims=True)
    x = (x1 - mu) / jnp.sqrt(var + 1e-5) * b8_ref[2:3, :] + b8_ref[3:4, :]
    t = jnp.dot(_bf(x), w1_ref[...], preferred_element_type=jnp.float32)
    t = jnp.maximum(t + fb1_ref[0:1, :], 0.0)
    y = jnp.dot(_bf(t), w2_ref[...], preferred_element_type=jnp.float32)
    y = y + b8_ref[6:7, :]
    x2 = x + y * res
    mu2 = jnp.mean(x2, axis=1, keepdims=True)
    var2 = jnp.mean((x2 - mu2) ** 2, axis=1, keepdims=True)
    out_ref[...] = ((x2 - mu2) / jnp.sqrt(var2 + 1e-5) * b8_ref[4:5, :]
                    + b8_ref[5:6, :])


def _row_spec():
    return pl.BlockSpec((BLK, D), lambda i: (i, 0))


def _full_spec(shape):
    nd = len(shape)
    return pl.BlockSpec(shape, lambda i: (0,) * nd)


def kernel(feat, edge_index, fc_w, fc_b, q_w, q_b, k_w, k_b, v_w, v_b,
           attn_scale, attn_bias, res_scale, ln1_g, ln1_b, ln2_g, ln2_b,
           ffn_w1, ffn_b1, ffn_w2, ffn_b2):
    feat_p = jnp.pad(feat, ((0, NP - N), (0, 0)))
    b2 = jnp.zeros((8, D), jnp.float32).at[0].set(fc_b)

    h_p = pl.pallas_call(
        _tca_body,
        grid=(GRID,),
        in_specs=[_row_spec(), _full_spec((D, D)), _full_spec((8, D))],
        out_specs=_row_spec(),
        out_shape=jax.ShapeDtypeStruct((NP, D), jnp.float32),
        interpret=_INTERPRET,
    )(feat_p.astype(jnp.bfloat16), fc_w.astype(jnp.bfloat16), b2)

    src = edge_index[0]
    dst = edge_index[1]
    mesh = plsc.VectorSubcoreMesh(core_axis_name="c", subcore_axis_name="s")

    scp = pltpu.CompilerParams(needs_layout_passes=False)
    sc1 = functools.partial(
        pl.kernel, mesh=mesh, compiler_params=scp,
        out_type=jax.ShapeDtypeStruct((NW * NP,), jnp.int32),
        scratch_types=[pltpu.VMEM((EPW,), jnp.int32),
                       pltpu.VMEM((NP,), jnp.int32),
                       pltpu.SemaphoreType.DMA],
    )(_sc_winner_partials)
    partials = sc1(dst)

    sc2 = functools.partial(
        pl.kernel, mesh=mesh, compiler_params=scp,
        out_type=[jax.ShapeDtypeStruct((NP,), jnp.float32),
                  jax.ShapeDtypeStruct((NP, D), jnp.float32)],
        scratch_types=[pltpu.VMEM((NW * NPW,), jnp.int32),
                       pltpu.VMEM((NPW,), jnp.int32),
                       pltpu.VMEM((NPW,), jnp.int32),
                       pltpu.VMEM((NPW,), jnp.float32),
                       pltpu.VMEM((NPW // 4, D), jnp.float32),
                       pltpu.VMEM((NPW // 4, D), jnp.float32),
                       pltpu.SemaphoreType.DMA,
                       pltpu.SemaphoreType.DMA],
    )(_sc_merge_gather)
    m1d, hg = sc2(partials, src, h_p)
    m = m1d.reshape(NP, 1)

    bt = jnp.zeros((D, 8), jnp.float32)
    bt = bt.at[:, 0].set(q_b).at[:, 1].set(k_b).at[:, 2].set(v_b)
    b8 = (jnp.zeros((8, D), jnp.float32)
          .at[2].set(ln1_g).at[3].set(ln1_b)
          .at[4].set(ln2_g).at[5].set(ln2_b)
          .at[6].set(ffn_b2))
    fb1 = jnp.zeros((8, DFF), jnp.float32).at[0].set(ffn_b1)
    cstt = jnp.zeros((16, 128), jnp.float32)
    cstt = cstt.at[0:H, 0:H].set(attn_bias[:, :, 0].T)
    cstt = cstt.at[13, 0].set(attn_scale / math.sqrt(DH))
    cstt = cstt.at[13, 1].set(res_scale)

    out_p = pl.pallas_call(
        _tcb_body,
        grid=(GRID,),
        in_specs=[_row_spec(), _row_spec(),
                  pl.BlockSpec((BLK, 1), lambda i: (i, 0)),
                  _full_spec((D, D)), _full_spec((D, D)), _full_spec((D, D)),
                  _full_spec((D, 8)),
                  _full_spec((8, D)), _full_spec((8, DFF)),
                  _full_spec((16, 128)),
                  _full_spec((D, DFF)), _full_spec((DFF, D))],
        out_specs=_row_spec(),
        out_shape=jax.ShapeDtypeStruct((NP, D), jnp.float32),
        scratch_shapes=[pltpu.VMEM((D, BLK), jnp.float32)],
        interpret=_INTERPRET,
    )(h_p, hg, m, q_w.T.astype(jnp.bfloat16), k_w.T.astype(jnp.bfloat16),
      v_w.T.astype(jnp.bfloat16), bt, b8, fb1, cstt,
      ffn_w1.astype(jnp.bfloat16), ffn_w2.astype(jnp.bfloat16))

    return out_p[:N]


# speedup vs baseline: 1.0694x; 1.0450x over previous
import functools
import math

import jax
import jax.numpy as jnp
import numpy as np
from jax import lax
from jax.experimental import pallas as pl
from jax.experimental.pallas import tpu as pltpu
from jax.experimental.pallas import tpu_sc as plsc

N = 10000
E = 160000
D = 384
H = 12
DH = 32
DFF = 4 * D
NP = 10240
BLK = 256
GRID = NP // BLK

NW = 32
EPW = E // NW
NPW = NP // NW
_SENT = np.uint32(0xFFFFFFFF)

_INTERPRET = False


def _lane_shift_by(x, r):
    idx = jnp.minimum(lax.iota(jnp.int32, 16) + r, 15)
    return lax.gather(
        x, idx[:, None],
        lax.GatherDimensionNumbers(offset_dims=(), collapsed_slice_dims=(0,),
                                   start_index_map=(0,)),
        (1,), mode=lax.GatherScatterMode.PROMISE_IN_BOUNDS)


def _sc_winner_partials(dst_hbm, part_hbm, dstv, wloc, sem):
    wid = lax.axis_index("s") * 2 + lax.axis_index("c")
    base = wid * EPW
    pltpu.sync_copy(dst_hbm.at[pl.ds(base * 1, EPW)], dstv)

    def init_body(i, _):
        st = pl.multiple_of(i * 16, 16)
        wloc[pl.ds(st, 16)] = jnp.full((16,), -1, jnp.int32)
        return 0
    lax.fori_loop(0, NP // 16, init_body, 0)

    iota = lax.iota(jnp.int32, 16)
    nvec = (EPW + 15) // 16

    def body(i, _):
        st = pl.multiple_of(i * 16, 16)
        d16 = dstv[pl.ds(st, 16)]
        lane_e = i * 16 + iota
        valid = lane_e < EPW
        es = base + lane_e
        conflict = iota < 0
        for r in range(1, 16):
            shifted = _lane_shift_by(d16, r)
            conflict = conflict | ((shifted == d16) & (iota < 16 - r))
        ok = valid & jnp.logical_not(conflict)
        plsc.store_scatter(wloc, [d16], es, mask=ok)
        return 0
    lax.fori_loop(0, nvec, body, 0)
    handles = []
    for s in range(NW):
        handles.append(pltpu.async_copy(
            wloc.at[pl.ds(s * NPW, NPW)],
            part_hbm.at[pl.ds((s * NW + wid) * NPW, NPW)], sem))
    for hd in handles:
        hd.wait()


def _sc_merge_gather(part_hbm, src_hbm, h_hbm, m_hbm, hg_hbm,
                     pblk, winc, srcv, mbuf, bufa, bufb, sema, semb):
    wid = lax.axis_index("s") * 2 + lax.axis_index("c")
    nbase = wid * NPW
    pltpu.sync_copy(part_hbm.at[pl.ds(wid * (NW * NPW), NW * NPW)], pblk)

    def merge_body(j, _):
        st = pl.multiple_of(j * 16, 16)
        mx = pblk[pl.ds(st, 16)]
        for r in range(1, NW):
            mx = jnp.maximum(mx, pblk[pl.ds(r * NPW + st, 16)])
        winc[pl.ds(st, 16)] = jnp.maximum(mx, 0)
        mbuf[pl.ds(st, 16)] = jnp.where(mx >= 0, 1.0, 0.0).astype(jnp.float32)
        return 0
    lax.fori_loop(0, NPW // 16, merge_body, 0)
    pltpu.sync_copy(mbuf, m_hbm.at[pl.ds(nbase * 1, NPW)])
    pltpu.async_copy(src_hbm.at[winc], srcv, sema).wait()

    ck = NPW // 4
    cps = {}
    cps[0] = pltpu.async_copy(h_hbm.at[srcv.at[pl.ds(0 * ck, ck)]], bufa, sema)
    cps[1] = pltpu.async_copy(h_hbm.at[srcv.at[pl.ds(1 * ck, ck)]], bufb, semb)
    for c in range(4):
        buf = bufa if c % 2 == 0 else bufb
        sem = sema if c % 2 == 0 else semb
        cps[c].wait()
        pltpu.sync_copy(buf, hg_hbm.at[pl.ds(nbase + c * ck, ck)])
        if c + 2 < 4:
            cps[c + 2] = pltpu.async_copy(
                h_hbm.at[srcv.at[pl.ds((c + 2) * ck, ck)]], buf, sem)


def _tca_body(feat_ref, fcw_ref, b2_ref, h_ref):
    f = feat_ref[...]
    h = jnp.dot(f, fcw_ref[...], preferred_element_type=jnp.float32)
    h_ref[...] = h + b2_ref[0:1, :]


def _bf(x):
    return x.astype(jnp.bfloat16)


def _tcb_body(h_ref, hg_ref, m_ref, qwt_ref, kwt_ref, vwt_ref, bt_ref,
              b8_ref, fb1_ref, cstt_ref, w1_ref, w2_ref, out_ref,
              hpt_ref, xs_ref):
    eff = cstt_ref[13:14, 0:1]
    res = cstt_ref[13:14, 1:2]

    xp = xs_ref[...]
    t = jnp.dot(_bf(xp), w1_ref[...], preferred_element_type=jnp.float32)
    t = jnp.maximum(t + fb1_ref[0:1, :], 0.0)
    y = jnp.dot(_bf(t), w2_ref[...], preferred_element_type=jnp.float32)
    y = y + b8_ref[6:7, :]
    x2 = xp + y * res
    mu2 = jnp.mean(x2, axis=1, keepdims=True)
    var2 = jnp.mean((x2 - mu2) ** 2, axis=1, keepdims=True)
    out_ref[...] = ((x2 - mu2) / jnp.sqrt(var2 + 1e-5) * b8_ref[4:5, :]
                    + b8_ref[5:6, :])

    hb = h_ref[...]
    hg = hg_ref[...]
    htb = _bf(hb)
    hgtb = _bf(hg)
    dnt = (((1,), (1,)), ((), ()))
    qt = lax.dot_general(qwt_ref[...], htb, dnt,
                         preferred_element_type=jnp.float32)
    qt = qt + bt_ref[:, 0:1]
    kt = lax.dot_general(kwt_ref[...], hgtb, dnt,
                         preferred_element_type=jnp.float32)
    kt = kt + bt_ref[:, 1:2]
    vt = lax.dot_general(vwt_ref[...], hgtb, dnt,
                         preferred_element_type=jnp.float32)
    vt = vt + bt_ref[:, 2:3]

    for hh in range(H):
        qh = qt[hh * DH:(hh + 1) * DH, :]
        ss = []
        for g in range(H):
            kg = kt[g * DH:(g + 1) * DH, :]
            ss.append(jnp.sum(qh * kg, axis=0, keepdims=True))
        s = jnp.concatenate(ss, axis=0)
        s = s * eff + cstt_ref[0:H, hh:hh + 1]
        smax = jnp.max(s, axis=0, keepdims=True)
        e = jnp.exp(s - smax)
        rs = 1.0 / jnp.sum(e, axis=0, keepdims=True)
        oh = e[0:1, :] * vt[0:DH, :]
        for g in range(1, H):
            oh = oh + e[g:g + 1, :] * vt[g * DH:(g + 1) * DH, :]
        hpt_ref[hh * DH:(hh + 1) * DH, :] = oh * rs
    h_part = hpt_ref[...].T

    m = m_ref[...]
    h_upd = h_part * m + hb * (1.0 - m)
    x1 = hb + h_upd * res
    mu = jnp.mean(x1, axis=1, keepdims=True)
    var = jnp.mean((x1 - mu) ** 2, axis=1, keepdims=True)
    xs_ref[...] = ((x1 - mu) / jnp.sqrt(var + 1e-5) * b8_ref[2:3, :]
                   + b8_ref[3:4, :])


def _row_spec():
    return pl.BlockSpec((BLK, D), lambda i: (i, 0))


def _pipe_in_spec():
    return pl.BlockSpec((BLK, D), lambda i: (jnp.minimum(i, GRID - 1), 0))


def _full_spec(shape):
    nd = len(shape)
    return pl.BlockSpec(shape, lambda i: (0,) * nd)


def kernel(feat, edge_index, fc_w, fc_b, q_w, q_b, k_w, k_b, v_w, v_b,
           attn_scale, attn_bias, res_scale, ln1_g, ln1_b, ln2_g, ln2_b,
           ffn_w1, ffn_b1, ffn_w2, ffn_b2):
    feat_p = jnp.pad(feat, ((0, NP - N), (0, 0)))
    b2 = jnp.zeros((8, D), jnp.float32).at[0].set(fc_b)

    h_p = pl.pallas_call(
        _tca_body,
        grid=(GRID,),
        in_specs=[_row_spec(), _full_spec((D, D)), _full_spec((8, D))],
        out_specs=_row_spec(),
        out_shape=jax.ShapeDtypeStruct((NP, D), jnp.float32),
        interpret=_INTERPRET,
    )(feat_p.astype(jnp.bfloat16), fc_w.astype(jnp.bfloat16), b2)

    src = edge_index[0]
    dst = edge_index[1]
    mesh = plsc.VectorSubcoreMesh(core_axis_name="c", subcore_axis_name="s")

    scp = pltpu.CompilerParams(needs_layout_passes=False)
    sc1 = functools.partial(
        pl.kernel, mesh=mesh, compiler_params=scp,
        out_type=jax.ShapeDtypeStruct((NW * NP,), jnp.int32),
        scratch_types=[pltpu.VMEM((EPW,), jnp.int32),
                       pltpu.VMEM((NP,), jnp.int32),
                       pltpu.SemaphoreType.DMA],
    )(_sc_winner_partials)
    partials = sc1(dst)

    sc2 = functools.partial(
        pl.kernel, mesh=mesh, compiler_params=scp,
        out_type=[jax.ShapeDtypeStruct((NP,), jnp.float32),
                  jax.ShapeDtypeStruct((NP, D), jnp.float32)],
        scratch_types=[pltpu.VMEM((NW * NPW,), jnp.int32),
                       pltpu.VMEM((NPW,), jnp.int32),
                       pltpu.VMEM((NPW,), jnp.int32),
                       pltpu.VMEM((NPW,), jnp.float32),
                       pltpu.VMEM((NPW // 4, D), jnp.float32),
                       pltpu.VMEM((NPW // 4, D), jnp.float32),
                       pltpu.SemaphoreType.DMA,
                       pltpu.SemaphoreType.DMA],
    )(_sc_merge_gather)
    m1d, hg = sc2(partials, src, h_p)
    m = m1d.reshape(NP, 1)

    bt = jnp.zeros((D, 8), jnp.float32)
    bt = bt.at[:, 0].set(q_b).at[:, 1].set(k_b).at[:, 2].set(v_b)
    b8 = (jnp.zeros((8, D), jnp.float32)
          .at[2].set(ln1_g).at[3].set(ln1_b)
          .at[4].set(ln2_g).at[5].set(ln2_b)
          .at[6].set(ffn_b2))
    fb1 = jnp.zeros((8, DFF), jnp.float32).at[0].set(ffn_b1)
    cstt = jnp.zeros((16, 128), jnp.float32)
    cstt = cstt.at[0:H, 0:H].set(attn_bias[:, :, 0].T)
    cstt = cstt.at[13, 0].set(attn_scale / math.sqrt(DH))
    cstt = cstt.at[13, 1].set(res_scale)

    out_p = pl.pallas_call(
        _tcb_body,
        grid=(GRID + 1,),
        in_specs=[_pipe_in_spec(), _pipe_in_spec(),
                  pl.BlockSpec((BLK, 1), lambda i: (jnp.minimum(i, GRID - 1), 0)),
                  _full_spec((D, D)), _full_spec((D, D)), _full_spec((D, D)),
                  _full_spec((D, 8)),
                  _full_spec((8, D)), _full_spec((8, DFF)),
                  _full_spec((16, 128)),
                  _full_spec((D, DFF)), _full_spec((DFF, D))],
        out_specs=pl.BlockSpec((BLK, D), lambda i: (jnp.maximum(i - 1, 0), 0)),
        out_shape=jax.ShapeDtypeStruct((NP, D), jnp.float32),
        scratch_shapes=[pltpu.VMEM((D, BLK), jnp.float32),
                        pltpu.VMEM((BLK, D), jnp.float32)],
        interpret=_INTERPRET,
    )(h_p, hg, m, q_w.T.astype(jnp.bfloat16), k_w.T.astype(jnp.bfloat16),
      v_w.T.astype(jnp.bfloat16), bt, b8, fb1, cstt,
      ffn_w1.astype(jnp.bfloat16), ffn_w2.astype(jnp.bfloat16))

    return out_p[:N]


# speedup vs baseline: 1.1648x; 1.0892x over previous
import functools
import math

import jax
import jax.numpy as jnp
import numpy as np
from jax import lax
from jax.experimental import pallas as pl
from jax.experimental.pallas import tpu as pltpu
from jax.experimental.pallas import tpu_sc as plsc

N = 10000
E = 160000
D = 384
H = 12
DH = 32
DFF = 4 * D
NP = 10240
BLK = 256
GRID = NP // BLK

NW = 32
EPW = E // NW
NPW = NP // NW
_SENT = np.uint32(0xFFFFFFFF)

_INTERPRET = False


def _lane_shift_by(x, r):
    idx = jnp.minimum(lax.iota(jnp.int32, 16) + r, 15)
    return lax.gather(
        x, idx[:, None],
        lax.GatherDimensionNumbers(offset_dims=(), collapsed_slice_dims=(0,),
                                   start_index_map=(0,)),
        (1,), mode=lax.GatherScatterMode.PROMISE_IN_BOUNDS)


def _sc_winner_partials(dst_hbm, part_hbm, dstv, wloc, sem):
    wid = lax.axis_index("s") * 2 + lax.axis_index("c")
    base = wid * EPW
    pltpu.sync_copy(dst_hbm.at[pl.ds(base * 1, EPW)], dstv)

    def init_body(i, _):
        st = pl.multiple_of(i * 16, 16)
        wloc[pl.ds(st, 16)] = jnp.full((16,), -1, jnp.int32)
        return 0
    lax.fori_loop(0, NP // 16, init_body, 0)

    iota = lax.iota(jnp.int32, 16)
    nvec = (EPW + 15) // 16

    def body(i, _):
        st = pl.multiple_of(i * 16, 16)
        d16 = dstv[pl.ds(st, 16)]
        lane_e = i * 16 + iota
        valid = lane_e < EPW
        es = base + lane_e
        conflict = iota < 0
        for r in range(1, 16):
            shifted = _lane_shift_by(d16, r)
            conflict = conflict | ((shifted == d16) & (iota < 16 - r))
        ok = valid & jnp.logical_not(conflict)
        plsc.store_scatter(wloc, [d16], es, mask=ok)
        return 0
    lax.fori_loop(0, nvec, body, 0)
    handles = []
    for s in range(NW):
        handles.append(pltpu.async_copy(
            wloc.at[pl.ds(s * NPW, NPW)],
            part_hbm.at[pl.ds((s * NW + wid) * NPW, NPW)], sem))
    for hd in handles:
        hd.wait()


def _sc_merge_gather(part_hbm, src_hbm, h_hbm, m_hbm, hg_hbm,
                     pblk, winc, srcv, mbuf, bufa, bufb, sema, semb):
    wid = lax.axis_index("s") * 2 + lax.axis_index("c")
    nbase = wid * NPW
    pltpu.sync_copy(part_hbm.at[pl.ds(wid * (NW * NPW), NW * NPW)], pblk)

    def merge_body(j, _):
        st = pl.multiple_of(j * 16, 16)
        mx = pblk[pl.ds(st, 16)]
        for r in range(1, NW):
            mx = jnp.maximum(mx, pblk[pl.ds(r * NPW + st, 16)])
        winc[pl.ds(st, 16)] = jnp.maximum(mx, 0)
        mbuf[pl.ds(st, 16)] = jnp.where(mx >= 0, 1.0, 0.0).astype(jnp.float32)
        return 0
    lax.fori_loop(0, NPW // 16, merge_body, 0)
    pltpu.sync_copy(mbuf, m_hbm.at[pl.ds(nbase * 1, NPW)])
    pltpu.async_copy(src_hbm.at[winc], srcv, sema).wait()

    ck = NPW // 4
    cps = {}
    cps[0] = pltpu.async_copy(h_hbm.at[srcv.at[pl.ds(0 * ck, ck)]], bufa, sema)
    cps[1] = pltpu.async_copy(h_hbm.at[srcv.at[pl.ds(1 * ck, ck)]], bufb, semb)
    for c in range(4):
        buf = bufa if c % 2 == 0 else bufb
        sem = sema if c % 2 == 0 else semb
        cps[c].wait()
        pltpu.sync_copy(buf, hg_hbm.at[pl.ds(nbase + c * ck, ck)])
        if c + 2 < 4:
            cps[c + 2] = pltpu.async_copy(
                h_hbm.at[srcv.at[pl.ds((c + 2) * ck, ck)]], buf, sem)


def _tca_body(feat_ref, fcw_ref, b2_ref, h_ref):
    f = feat_ref[...]
    h = jnp.dot(_bf(f), fcw_ref[...], preferred_element_type=jnp.float32)
    h_ref[...] = h + b2_ref[0:1, :]


def _bf(x):
    return x.astype(jnp.bfloat16)


def _tcb_body(h_ref, hg_ref, m_ref, qwt_ref, kwt_ref, vwt_ref, bt_ref,
              b8_ref, fb1_ref, cstt_ref, w1_ref, w2_ref, out_ref,
              hpt_ref, xs_ref):
    eff = cstt_ref[13:14, 0:1]
    res = cstt_ref[13:14, 1:2]

    xp = xs_ref[...]
    t = jnp.dot(_bf(xp), w1_ref[...], preferred_element_type=jnp.float32)
    t = jnp.maximum(t + fb1_ref[0:1, :], 0.0)
    y = jnp.dot(_bf(t), w2_ref[...], preferred_element_type=jnp.float32)
    y = y + b8_ref[6:7, :]
    x2 = xp + y * res
    mu2 = jnp.mean(x2, axis=1, keepdims=True)
    var2 = jnp.mean((x2 - mu2) ** 2, axis=1, keepdims=True)
    out_ref[...] = ((x2 - mu2) / jnp.sqrt(var2 + 1e-5) * b8_ref[4:5, :]
                    + b8_ref[5:6, :])

    hb = h_ref[...]
    hg = hg_ref[...]
    htb = _bf(hb)
    hgtb = _bf(hg)
    dnt = (((1,), (1,)), ((), ()))
    qt = lax.dot_general(qwt_ref[...], htb, dnt,
                         preferred_element_type=jnp.float32)
    qt = qt + bt_ref[:, 0:1]
    kt = lax.dot_general(kwt_ref[...], hgtb, dnt,
                         preferred_element_type=jnp.float32)
    kt = kt + bt_ref[:, 1:2]
    vt = lax.dot_general(vwt_ref[...], hgtb, dnt,
                         preferred_element_type=jnp.float32)
    vt = vt + bt_ref[:, 2:3]

    for hh in range(H):
        qh = qt[hh * DH:(hh + 1) * DH, :]
        ss = []
        for g in range(H):
            kg = kt[g * DH:(g + 1) * DH, :]
            ss.append(jnp.sum(qh * kg, axis=0, keepdims=True))
        s = jnp.concatenate(ss, axis=0)
        s = s * eff + cstt_ref[0:H, hh:hh + 1]
        smax = jnp.max(s, axis=0, keepdims=True)
        e = jnp.exp(s - smax)
        rs = 1.0 / jnp.sum(e, axis=0, keepdims=True)
        oh = e[0:1, :] * vt[0:DH, :]
        for g in range(1, H):
            oh = oh + e[g:g + 1, :] * vt[g * DH:(g + 1) * DH, :]
        hpt_ref[hh * DH:(hh + 1) * DH, :] = oh * rs
    h_part = hpt_ref[...].T

    m = m_ref[...]
    h_upd = h_part * m + hb * (1.0 - m)
    x1 = hb + h_upd * res
    mu = jnp.mean(x1, axis=1, keepdims=True)
    var = jnp.mean((x1 - mu) ** 2, axis=1, keepdims=True)
    xs_ref[...] = ((x1 - mu) / jnp.sqrt(var + 1e-5) * b8_ref[2:3, :]
                   + b8_ref[3:4, :])


def _row_spec():
    return pl.BlockSpec((BLK, D), lambda i: (i, 0))


def _pipe_in_spec():
    return pl.BlockSpec((BLK, D), lambda i: (jnp.minimum(i, GRID - 1), 0))


def _full_spec(shape):
    nd = len(shape)
    return pl.BlockSpec(shape, lambda i: (0,) * nd)


def kernel(feat, edge_index, fc_w, fc_b, q_w, q_b, k_w, k_b, v_w, v_b,
           attn_scale, attn_bias, res_scale, ln1_g, ln1_b, ln2_g, ln2_b,
           ffn_w1, ffn_b1, ffn_w2, ffn_b2):
    b2 = jnp.zeros((8, D), jnp.float32).at[0].set(fc_b)

    h_p = pl.pallas_call(
        _tca_body,
        grid=(GRID,),
        in_specs=[_row_spec(), _full_spec((D, D)), _full_spec((8, D))],
        out_specs=_row_spec(),
        out_shape=jax.ShapeDtypeStruct((NP, D), jnp.float32),
        interpret=_INTERPRET,
    )(feat, fc_w.astype(jnp.bfloat16), b2)

    src = edge_index[0]
    dst = edge_index[1]
    mesh = plsc.VectorSubcoreMesh(core_axis_name="c", subcore_axis_name="s")

    scp = pltpu.CompilerParams(needs_layout_passes=False)
    sc1 = functools.partial(
        pl.kernel, mesh=mesh, compiler_params=scp,
        out_type=jax.ShapeDtypeStruct((NW * NP,), jnp.int32),
        scratch_types=[pltpu.VMEM((EPW,), jnp.int32),
                       pltpu.VMEM((NP,), jnp.int32),
                       pltpu.SemaphoreType.DMA],
    )(_sc_winner_partials)
    partials = sc1(dst)

    sc2 = functools.partial(
        pl.kernel, mesh=mesh, compiler_params=scp,
        out_type=[jax.ShapeDtypeStruct((NP,), jnp.float32),
                  jax.ShapeDtypeStruct((NP, D), jnp.float32)],
        scratch_types=[pltpu.VMEM((NW * NPW,), jnp.int32),
                       pltpu.VMEM((NPW,), jnp.int32),
                       pltpu.VMEM((NPW,), jnp.int32),
                       pltpu.VMEM((NPW,), jnp.float32),
                       pltpu.VMEM((NPW // 4, D), jnp.float32),
                       pltpu.VMEM((NPW // 4, D), jnp.float32),
                       pltpu.SemaphoreType.DMA,
                       pltpu.SemaphoreType.DMA],
    )(_sc_merge_gather)
    m1d, hg = sc2(partials, src, h_p)
    m = m1d.reshape(NP, 1)

    bt = jnp.zeros((D, 8), jnp.float32)
    bt = bt.at[:, 0].set(q_b).at[:, 1].set(k_b).at[:, 2].set(v_b)
    b8 = (jnp.zeros((8, D), jnp.float32)
          .at[2].set(ln1_g).at[3].set(ln1_b)
          .at[4].set(ln2_g).at[5].set(ln2_b)
          .at[6].set(ffn_b2))
    fb1 = jnp.zeros((8, DFF), jnp.float32).at[0].set(ffn_b1)
    cstt = jnp.zeros((16, 128), jnp.float32)
    cstt = cstt.at[0:H, 0:H].set(attn_bias[:, :, 0].T)
    cstt = cstt.at[13, 0].set(attn_scale / math.sqrt(DH))
    cstt = cstt.at[13, 1].set(res_scale)

    out_p = pl.pallas_call(
        _tcb_body,
        grid=(GRID + 1,),
        in_specs=[_pipe_in_spec(), _pipe_in_spec(),
                  pl.BlockSpec((BLK, 1), lambda i: (jnp.minimum(i, GRID - 1), 0)),
                  _full_spec((D, D)), _full_spec((D, D)), _full_spec((D, D)),
                  _full_spec((D, 8)),
                  _full_spec((8, D)), _full_spec((8, DFF)),
                  _full_spec((16, 128)),
                  _full_spec((D, DFF)), _full_spec((DFF, D))],
        out_specs=pl.BlockSpec((BLK, D), lambda i: (jnp.maximum(i - 1, 0), 0)),
        out_shape=jax.ShapeDtypeStruct((N, D), jnp.float32),
        scratch_shapes=[pltpu.VMEM((D, BLK), jnp.float32),
                        pltpu.VMEM((BLK, D), jnp.float32)],
        interpret=_INTERPRET,
    )(h_p, hg, m, q_w.T.astype(jnp.bfloat16), k_w.T.astype(jnp.bfloat16),
      v_w.T.astype(jnp.bfloat16), bt, b8, fb1, cstt,
      ffn_w1.astype(jnp.bfloat16), ffn_w2.astype(jnp.bfloat16))

    return out_p


# speedup vs baseline: 1.2568x; 1.0790x over previous
import functools
import math

import jax
import jax.numpy as jnp
import numpy as np
from jax import lax
from jax.experimental import pallas as pl
from jax.experimental.pallas import tpu as pltpu
from jax.experimental.pallas import tpu_sc as plsc

N = 10000
E = 160000
D = 384
H = 12
DH = 32
DFF = 4 * D
NP = 10240
BLK = 512
GRID = NP // BLK

NW = 32
EPW = E // NW
NPW = NP // NW
_SENT = np.uint32(0xFFFFFFFF)

_INTERPRET = False


def _lane_shift_by(x, r):
    idx = jnp.minimum(lax.iota(jnp.int32, 16) + r, 15)
    return lax.gather(
        x, idx[:, None],
        lax.GatherDimensionNumbers(offset_dims=(), collapsed_slice_dims=(0,),
                                   start_index_map=(0,)),
        (1,), mode=lax.GatherScatterMode.PROMISE_IN_BOUNDS)


def _sc_winner_partials(dst_hbm, part_hbm, dstv, wloc, sem):
    wid = lax.axis_index("s") * 2 + lax.axis_index("c")
    base = wid * EPW
    pltpu.sync_copy(dst_hbm.at[pl.ds(base * 1, EPW)], dstv)

    def init_body(i, _):
        st = pl.multiple_of(i * 16, 16)
        wloc[pl.ds(st, 16)] = jnp.full((16,), -1, jnp.int32)
        return 0
    lax.fori_loop(0, NP // 16, init_body, 0)

    iota = lax.iota(jnp.int32, 16)
    nvec = (EPW + 15) // 16

    def body(i, _):
        st = pl.multiple_of(i * 16, 16)
        d16 = dstv[pl.ds(st, 16)]
        lane_e = i * 16 + iota
        valid = lane_e < EPW
        es = base + lane_e
        conflict = iota < 0
        for r in range(1, 16):
            shifted = _lane_shift_by(d16, r)
            conflict = conflict | ((shifted == d16) & (iota < 16 - r))
        ok = valid & jnp.logical_not(conflict)
        plsc.store_scatter(wloc, [d16], es, mask=ok)
        return 0
    lax.fori_loop(0, nvec, body, 0)
    handles = []
    for s in range(NW):
        handles.append(pltpu.async_copy(
            wloc.at[pl.ds(s * NPW, NPW)],
            part_hbm.at[pl.ds((s * NW + wid) * NPW, NPW)], sem))
    for hd in handles:
        hd.wait()


def _sc_merge_gather(part_hbm, src_hbm, h_hbm, m_hbm, hg_hbm,
                     pblk, winc, srcv, mbuf, bufa, bufb, sema, semb):
    wid = lax.axis_index("s") * 2 + lax.axis_index("c")
    nbase = wid * NPW
    pltpu.sync_copy(part_hbm.at[pl.ds(wid * (NW * NPW), NW * NPW)], pblk)

    def merge_body(j, _):
        st = pl.multiple_of(j * 16, 16)
        mx = pblk[pl.ds(st, 16)]
        for r in range(1, NW):
            mx = jnp.maximum(mx, pblk[pl.ds(r * NPW + st, 16)])
        winc[pl.ds(st, 16)] = jnp.maximum(mx, 0)
        mbuf[pl.ds(st, 16)] = jnp.where(mx >= 0, 1.0, 0.0).astype(jnp.float32)
        return 0
    lax.fori_loop(0, NPW // 16, merge_body, 0)
    pltpu.sync_copy(mbuf, m_hbm.at[pl.ds(nbase * 1, NPW)])
    pltpu.async_copy(src_hbm.at[winc], srcv, sema).wait()

    ck = NPW // 4
    cps = {}
    cps[0] = pltpu.async_copy(h_hbm.at[srcv.at[pl.ds(0 * ck, ck)]], bufa, sema)
    cps[1] = pltpu.async_copy(h_hbm.at[srcv.at[pl.ds(1 * ck, ck)]], bufb, semb)
    for c in range(4):
        buf = bufa if c % 2 == 0 else bufb
        sem = sema if c % 2 == 0 else semb
        cps[c].wait()
        pltpu.sync_copy(buf, hg_hbm.at[pl.ds(nbase + c * ck, ck)])
        if c + 2 < 4:
            cps[c + 2] = pltpu.async_copy(
                h_hbm.at[srcv.at[pl.ds((c + 2) * ck, ck)]], buf, sem)


def _tca_body(feat_ref, fcw_ref, b2_ref, h_ref):
    f = feat_ref[...]
    h = jnp.dot(_bf(f), fcw_ref[...], preferred_element_type=jnp.float32)
    h_ref[...] = h + b2_ref[0:1, :]


def _bf(x):
    return x.astype(jnp.bfloat16)


def _tcb_body(h_ref, hg_ref, m_ref, qwt_ref, kwt_ref, vwt_ref, bt_ref,
              b8_ref, fb1_ref, cstt_ref, w1_ref, w2_ref, out_ref,
              hpt_ref, xs_ref):
    eff = cstt_ref[13:14, 0:1]
    res = cstt_ref[13:14, 1:2]

    xp = xs_ref[...]
    t = jnp.dot(_bf(xp), w1_ref[...], preferred_element_type=jnp.float32)
    t = jnp.maximum(t + fb1_ref[0:1, :], 0.0)
    y = jnp.dot(_bf(t), w2_ref[...], preferred_element_type=jnp.float32)
    y = y + b8_ref[6:7, :]
    x2 = xp + y * res
    mu2 = jnp.mean(x2, axis=1, keepdims=True)
    var2 = jnp.mean((x2 - mu2) ** 2, axis=1, keepdims=True)
    out_ref[...] = ((x2 - mu2) / jnp.sqrt(var2 + 1e-5) * b8_ref[4:5, :]
                    + b8_ref[5:6, :])

    hb = h_ref[...]
    hg = hg_ref[...]
    htb = _bf(hb)
    hgtb = _bf(hg)
    dnt = (((1,), (1,)), ((), ()))
    qt = lax.dot_general(qwt_ref[...], htb, dnt,
                         preferred_element_type=jnp.float32)
    qt = qt + bt_ref[:, 0:1]
    kt = lax.dot_general(kwt_ref[...], hgtb, dnt,
                         preferred_element_type=jnp.float32)
    kt = kt + bt_ref[:, 1:2]
    vt = lax.dot_general(vwt_ref[...], hgtb, dnt,
                         preferred_element_type=jnp.float32)
    vt = vt + bt_ref[:, 2:3]

    for hh in range(H):
        qh = qt[hh * DH:(hh + 1) * DH, :]
        ss = []
        for g in range(H):
            kg = kt[g * DH:(g + 1) * DH, :]
            ss.append(jnp.sum(qh * kg, axis=0, keepdims=True))
        s = jnp.concatenate(ss, axis=0)
        s = s * eff + cstt_ref[0:H, hh:hh + 1]
        smax = jnp.max(s, axis=0, keepdims=True)
        e = jnp.exp(s - smax)
        rs = 1.0 / jnp.sum(e, axis=0, keepdims=True)
        oh = e[0:1, :] * vt[0:DH, :]
        for g in range(1, H):
            oh = oh + e[g:g + 1, :] * vt[g * DH:(g + 1) * DH, :]
        hpt_ref[hh * DH:(hh + 1) * DH, :] = oh * rs
    h_part = hpt_ref[...].T

    m = m_ref[...]
    h_upd = h_part * m + hb * (1.0 - m)
    x1 = hb + h_upd * res
    mu = jnp.mean(x1, axis=1, keepdims=True)
    var = jnp.mean((x1 - mu) ** 2, axis=1, keepdims=True)
    xs_ref[...] = ((x1 - mu) / jnp.sqrt(var + 1e-5) * b8_ref[2:3, :]
                   + b8_ref[3:4, :])


def _row_spec():
    return pl.BlockSpec((BLK, D), lambda i: (i, 0))


def _pipe_in_spec():
    return pl.BlockSpec((BLK, D), lambda i: (jnp.minimum(i, GRID - 1), 0))


def _full_spec(shape):
    nd = len(shape)
    return pl.BlockSpec(shape, lambda i: (0,) * nd)


def kernel(feat, edge_index, fc_w, fc_b, q_w, q_b, k_w, k_b, v_w, v_b,
           attn_scale, attn_bias, res_scale, ln1_g, ln1_b, ln2_g, ln2_b,
           ffn_w1, ffn_b1, ffn_w2, ffn_b2):
    b2 = jnp.zeros((8, D), jnp.float32).at[0].set(fc_b)

    h_p = pl.pallas_call(
        _tca_body,
        grid=(GRID,),
        in_specs=[_row_spec(), _full_spec((D, D)), _full_spec((8, D))],
        out_specs=_row_spec(),
        out_shape=jax.ShapeDtypeStruct((NP, D), jnp.float32),
        interpret=_INTERPRET,
    )(feat, fc_w.astype(jnp.bfloat16), b2)

    src = edge_index[0]
    dst = edge_index[1]
    mesh = plsc.VectorSubcoreMesh(core_axis_name="c", subcore_axis_name="s")

    scp = pltpu.CompilerParams(needs_layout_passes=False)
    sc1 = functools.partial(
        pl.kernel, mesh=mesh, compiler_params=scp,
        out_type=jax.ShapeDtypeStruct((NW * NP,), jnp.int32),
        scratch_types=[pltpu.VMEM((EPW,), jnp.int32),
                       pltpu.VMEM((NP,), jnp.int32),
                       pltpu.SemaphoreType.DMA],
    )(_sc_winner_partials)
    partials = sc1(dst)

    sc2 = functools.partial(
        pl.kernel, mesh=mesh, compiler_params=scp,
        out_type=[jax.ShapeDtypeStruct((NP,), jnp.float32),
                  jax.ShapeDtypeStruct((NP, D), jnp.float32)],
        scratch_types=[pltpu.VMEM((NW * NPW,), jnp.int32),
                       pltpu.VMEM((NPW,), jnp.int32),
                       pltpu.VMEM((NPW,), jnp.int32),
                       pltpu.VMEM((NPW,), jnp.float32),
                       pltpu.VMEM((NPW // 4, D), jnp.float32),
                       pltpu.VMEM((NPW // 4, D), jnp.float32),
                       pltpu.SemaphoreType.DMA,
                       pltpu.SemaphoreType.DMA],
    )(_sc_merge_gather)
    m1d, hg = sc2(partials, src, h_p)
    m = m1d.reshape(NP, 1)

    bt = jnp.zeros((D, 8), jnp.float32)
    bt = bt.at[:, 0].set(q_b).at[:, 1].set(k_b).at[:, 2].set(v_b)
    b8 = (jnp.zeros((8, D), jnp.float32)
          .at[2].set(ln1_g).at[3].set(ln1_b)
          .at[4].set(ln2_g).at[5].set(ln2_b)
          .at[6].set(ffn_b2))
    fb1 = jnp.zeros((8, DFF), jnp.float32).at[0].set(ffn_b1)
    cstt = jnp.zeros((16, 128), jnp.float32)
    cstt = cstt.at[0:H, 0:H].set(attn_bias[:, :, 0].T)
    cstt = cstt.at[13, 0].set(attn_scale / math.sqrt(DH))
    cstt = cstt.at[13, 1].set(res_scale)

    out_p = pl.pallas_call(
        _tcb_body,
        grid=(GRID + 1,),
        in_specs=[_pipe_in_spec(), _pipe_in_spec(),
                  pl.BlockSpec((BLK, 1), lambda i: (jnp.minimum(i, GRID - 1), 0)),
                  _full_spec((D, D)), _full_spec((D, D)), _full_spec((D, D)),
                  _full_spec((D, 8)),
                  _full_spec((8, D)), _full_spec((8, DFF)),
                  _full_spec((16, 128)),
                  _full_spec((D, DFF)), _full_spec((DFF, D))],
        out_specs=pl.BlockSpec((BLK, D), lambda i: (jnp.maximum(i - 1, 0), 0)),
        out_shape=jax.ShapeDtypeStruct((N, D), jnp.float32),
        scratch_shapes=[pltpu.VMEM((D, BLK), jnp.float32),
                        pltpu.VMEM((BLK, D), jnp.float32)],
        interpret=_INTERPRET,
    )(h_p, hg, m, q_w.T.astype(jnp.bfloat16), k_w.T.astype(jnp.bfloat16),
      v_w.T.astype(jnp.bfloat16), bt, b8, fb1, cstt,
      ffn_w1.astype(jnp.bfloat16), ffn_w2.astype(jnp.bfloat16))

    return out_p
